# baseline (speedup 1.0000x reference)
import functools

import jax
import jax.numpy as jnp
from jax import lax
from jax.experimental import pallas as pl
from jax.experimental.pallas import tpu as pltpu
from jax.experimental.pallas import tpu_sc as plsc

NC = 2
NS = 16
NW = NC * NS
B = 4096
D = 64
L = 50
MARGIN = 1.0
RPW = B // NW
KG = D // 16
NBUF = 2


def _tcf_body(uid_h, pid_h, nid_h, unbr_h, pnbr_h, nnbr_h, utab_h, itab_h,
              out_h,
              uidx_v, pidx_v, nidx_v, urows_v, prows_v, nrows_v,
              uni_v, pni_v, nni_v, ubm_v, pbm_v, nbm_v,
              jbu_v, jbp_v, jbn_v, out_v,
              ssem, jsem):
    wid = lax.axis_index("s") * NC + lax.axis_index("c")
    base = wid * RPW

    pltpu.sync_copy(uid_h.at[pl.ds(base, RPW)], uidx_v)
    pltpu.sync_copy(pid_h.at[pl.ds(base, RPW)], pidx_v)
    pltpu.sync_copy(nid_h.at[pl.ds(base, RPW)], nidx_v)
    cu = pltpu.async_copy(utab_h.at[uidx_v], urows_v, ssem)
    cp = pltpu.async_copy(itab_h.at[pidx_v], prows_v, ssem)
    cn = pltpu.async_copy(itab_h.at[nidx_v], nrows_v, ssem)
    pltpu.sync_copy(unbr_h.at[:, pl.ds(base, RPW)], uni_v)
    pltpu.sync_copy(pnbr_h.at[:, pl.ds(base, RPW)], pni_v)
    pltpu.sync_copy(nnbr_h.at[:, pl.ds(base, RPW)], nni_v)

    def start_j(j):
        slot = lax.rem(j, NBUF)
        pltpu.async_copy(itab_h.at[uni_v.at[j]], jbu_v.at[slot],
                         jsem.at[slot])
        pltpu.async_copy(utab_h.at[pni_v.at[j]], jbp_v.at[slot],
                         jsem.at[slot])
        pltpu.async_copy(utab_h.at[nni_v.at[j]], jbn_v.at[slot],
                         jsem.at[slot])

    def wait_j(j):
        slot = lax.rem(j, NBUF)
        pltpu.make_async_copy(itab_h.at[uni_v.at[j]], jbu_v.at[slot],
                              jsem.at[slot]).wait()
        pltpu.make_async_copy(utab_h.at[pni_v.at[j]], jbp_v.at[slot],
                              jsem.at[slot]).wait()
        pltpu.make_async_copy(utab_h.at[nni_v.at[j]], jbn_v.at[slot],
                              jsem.at[slot]).wait()

    for j in range(NBUF):
        start_j(j)

    def accum(slot, first):
        def body(r, _):
            for jb, acc in ((jbu_v, ubm_v), (jbp_v, pbm_v), (jbn_v, nbm_v)):
                for k in range(KG):
                    x = jb[slot, r, pl.ds(16 * k, 16)]
                    if first:
                        acc[r, pl.ds(16 * k, 16)] = x
                    else:
                        plsc.addupdate(acc.at[r, pl.ds(16 * k, 16)], x)
            return 0
        lax.fori_loop(0, RPW, body, 0)

    wait_j(0)
    accum(0, True)

    def j_body(j, _):
        @pl.when(j + (NBUF - 1) < L)
        def _():
            start_j(j + (NBUF - 1))
        wait_j(j)
        accum(lax.rem(j, NBUF), False)
        return 0

    lax.fori_loop(1, L, j_body, 0)

    cu.wait()
    cp.wait()
    cn.wait()

    inv_l = jnp.float32(1.0 / L)
    zero = jnp.zeros((16,), jnp.float32)

    def row_body(i, acc):
        new = []
        for k in range(KG):
            ub = ubm_v[i, pl.ds(16 * k, 16)] * inv_l
            pb = pbm_v[i, pl.ds(16 * k, 16)] * inv_l
            nb = nbm_v[i, pl.ds(16 * k, 16)] * inv_l
            u = urows_v[i, pl.ds(16 * k, 16)]
            pe = prows_v[i, pl.ds(16 * k, 16)]
            ne = nrows_v[i, pl.ds(16 * k, 16)]
            tpos = u + ub * pb - pe
            tneg = u + ub * nb - ne
            v = MARGIN + tpos * tpos - tneg * tneg
            new.append(acc[k] + jnp.maximum(v, 0.0))
        return tuple(new)

    acc = lax.fori_loop(0, RPW, row_body, (zero,) * KG)
    out_v[0, :] = acc[0] + acc[1] + acc[2] + acc[3]
    pltpu.sync_copy(out_v, out_h.at[pl.ds(wid, 1)])


def kernel(user_ids, pos_ids, neg_ids, user_nbr_items, pos_item_nbr_users,
           neg_item_nbr_users, user_table, item_table):
    unbr_t = user_nbr_items.T
    pnbr_t = pos_item_nbr_users.T
    nnbr_t = neg_item_nbr_users.T

    mesh = plsc.VectorSubcoreMesh(core_axis_name="c", subcore_axis_name="s")
    run = pl.kernel(
        _tcf_body,
        mesh=mesh,
        compiler_params=pltpu.CompilerParams(use_tc_tiling_on_sc=False),
        out_type=jax.ShapeDtypeStruct((NW, 16), jnp.float32),
        scratch_types=[
            pltpu.VMEM((RPW,), jnp.int32),
            pltpu.VMEM((RPW,), jnp.int32),
            pltpu.VMEM((RPW,), jnp.int32),
            pltpu.VMEM((RPW, D), jnp.float32),
            pltpu.VMEM((RPW, D), jnp.float32),
            pltpu.VMEM((RPW, D), jnp.float32),
            pltpu.VMEM((L, RPW), jnp.int32),
            pltpu.VMEM((L, RPW), jnp.int32),
            pltpu.VMEM((L, RPW), jnp.int32),
            pltpu.VMEM((RPW, D), jnp.float32),
            pltpu.VMEM((RPW, D), jnp.float32),
            pltpu.VMEM((RPW, D), jnp.float32),
            pltpu.VMEM((NBUF, RPW, D), jnp.float32),
            pltpu.VMEM((NBUF, RPW, D), jnp.float32),
            pltpu.VMEM((NBUF, RPW, D), jnp.float32),
            pltpu.VMEM((1, 16), jnp.float32),
            pltpu.SemaphoreType.DMA,
            pltpu.SemaphoreType.DMA((NBUF,)),
        ],
    )
    partials = run(user_ids, pos_ids, neg_ids, unbr_t, pnbr_t, nnbr_t,
                   user_table, item_table)
    return jnp.sum(partials)


# speedup vs baseline: 1.0070x; 1.0070x over previous
import functools

import jax
import jax.numpy as jnp
from jax import lax
from jax.experimental import pallas as pl
from jax.experimental.pallas import tpu as pltpu
from jax.experimental.pallas import tpu_sc as plsc

NC = 2
NS = 16
NW = NC * NS
B = 4096
D = 64
L = 50
MARGIN = 1.0
RPW = B // NW
HR = RPW // 2
KG = D // 16
NBUF = 2
NSTEP = L * 2


def _tcf_body(uid_h, pid_h, nid_h, unbr_h, pnbr_h, nnbr_h, utab_h, itab_h,
              out_h,
              uidx_v, pidx_v, nidx_v, urows_v, prows_v, nrows_v,
              uni_v, pni_v, nni_v, ubm_v, pbm_v, nbm_v,
              jbu_v, jbp_v, jbn_v, uring_v, pring_v, nring_v, out_v,
              ssem, jsem):
    wid = lax.axis_index("s") * NC + lax.axis_index("c")
    base = wid * RPW

    pltpu.sync_copy(uid_h.at[pl.ds(base, RPW)], uidx_v)
    pltpu.sync_copy(pid_h.at[pl.ds(base, RPW)], pidx_v)
    pltpu.sync_copy(nid_h.at[pl.ds(base, RPW)], nidx_v)
    pltpu.sync_copy(unbr_h.at[:, pl.ds(base, RPW)], uni_v)
    pltpu.sync_copy(pnbr_h.at[:, pl.ds(base, RPW)], pni_v)
    pltpu.sync_copy(nnbr_h.at[:, pl.ds(base, RPW)], nni_v)

    def singles(ids_v, tab_h, dest_v):
        for h in range(2):
            slot = h % NBUF
            for g in range(4):
                uring_v[slot, pl.ds(16 * g, 16)] = lax.shift_right_logical(
                    ids_v[pl.ds(HR * h + 16 * g, 16)], 1)
            pltpu.async_copy(tab_h.at[uring_v.at[slot]], jbu_v.at[slot],
                             ssem).wait()

            def compact(g, _):
                offv = (ids_v[pl.ds(HR * h + 16 * g, 16)] & 1) * D
                for r in range(16):
                    off = offv[r]
                    line = 32 * h + 8 * g + r // 2
                    doff = (r % 2) * D
                    for k in range(KG):
                        dest_v[line, pl.ds(doff + 16 * k, 16)] = (
                            jbu_v[slot, 16 * g + r, pl.ds(off + 16 * k, 16)])
                return 0

            lax.fori_loop(0, 4, compact, 0)

    singles(uidx_v, utab_h, urows_v)
    singles(pidx_v, itab_h, prows_v)
    singles(nidx_v, itab_h, nrows_v)

    zero = jnp.zeros((16,), jnp.float32)

    def zero_body(i, _):
        for acc_v in (ubm_v, pbm_v, nbm_v):
            for k in range(2 * KG):
                acc_v[i, pl.ds(16 * k, 16)] = zero
        return 0

    lax.fori_loop(0, HR, zero_body, 0)

    BAGS = ((uni_v, uring_v, jbu_v, ubm_v, itab_h),
            (pni_v, pring_v, jbp_v, pbm_v, utab_h),
            (nni_v, nring_v, jbn_v, nbm_v, utab_h))

    def start_step(s):
        j = s // 2
        h = lax.rem(s, 2)
        slot = lax.rem(s, NBUF)
        for ni_v, ring_v, jb_v, _, tab_h in BAGS:
            for g in range(4):
                ring_v[slot, pl.ds(16 * g, 16)] = lax.shift_right_logical(
                    ni_v[j, pl.ds(HR * h + 16 * g, 16)], 1)
            pltpu.async_copy(tab_h.at[ring_v.at[slot]], jb_v.at[slot],
                             jsem.at[slot])

    def wait_step(s):
        slot = lax.rem(s, NBUF)
        for ni_v, ring_v, jb_v, _, tab_h in BAGS:
            pltpu.make_async_copy(tab_h.at[ring_v.at[slot]], jb_v.at[slot],
                                  jsem.at[slot]).wait()

    def accum(s):
        j = s // 2
        h = lax.rem(s, 2)
        slot = lax.rem(s, NBUF)

        def body(g, _):
            for ni_v, ring_v, jb_v, acc_v, tab_h in BAGS:
                offv = (ni_v[j, pl.ds(HR * h + 16 * g, 16)] & 1) * D
                for r in range(16):
                    off = offv[r]
                    line = 32 * h + 8 * g + r // 2
                    doff = (r % 2) * D
                    for k in range(KG):
                        x = jb_v[slot, 16 * g + r, pl.ds(off + 16 * k, 16)]
                        plsc.addupdate(
                            acc_v.at[line, pl.ds(doff + 16 * k, 16)], x)
            return 0

        lax.fori_loop(0, 4, body, 0)

    start_step(0)

    def step_body(s, _):
        @pl.when(s + 1 < NSTEP)
        def _():
            start_step(s + 1)
        wait_step(s)
        accum(s)
        return 0

    lax.fori_loop(0, NSTEP, step_body, 0)

    inv_l = jnp.float32(1.0 / L)

    def row_body(i, acc):
        new = list(acc)
        for k in range(2 * KG):
            ub = ubm_v[i, pl.ds(16 * k, 16)] * inv_l
            pb = pbm_v[i, pl.ds(16 * k, 16)] * inv_l
            nb = nbm_v[i, pl.ds(16 * k, 16)] * inv_l
            u = urows_v[i, pl.ds(16 * k, 16)]
            pe = prows_v[i, pl.ds(16 * k, 16)]
            ne = nrows_v[i, pl.ds(16 * k, 16)]
            tpos = u + ub * pb - pe
            tneg = u + ub * nb - ne
            v = MARGIN + tpos * tpos - tneg * tneg
            new[k % KG] = new[k % KG] + jnp.maximum(v, 0.0)
        return tuple(new)

    acc = lax.fori_loop(0, HR, row_body, (zero,) * KG)
    out_v[0, pl.ds(0, 16)] = acc[0] + acc[1] + acc[2] + acc[3]
    for g in range(1, 8):
        out_v[0, pl.ds(16 * g, 16)] = zero
    pltpu.sync_copy(out_v, out_h.at[pl.ds(wid, 1)])


def kernel(user_ids, pos_ids, neg_ids, user_nbr_items, pos_item_nbr_users,
           neg_item_nbr_users, user_table, item_table):
    unbr_t = user_nbr_items.T
    pnbr_t = pos_item_nbr_users.T
    nnbr_t = neg_item_nbr_users.T
    utab2 = user_table.reshape(-1, 2 * D)
    itab2 = item_table.reshape(-1, 2 * D)

    mesh = plsc.VectorSubcoreMesh(core_axis_name="c", subcore_axis_name="s")
    run = pl.kernel(
        _tcf_body,
        mesh=mesh,
        compiler_params=pltpu.CompilerParams(use_tc_tiling_on_sc=True),
        out_type=jax.ShapeDtypeStruct((NW, 128), jnp.float32),
        scratch_types=[
            pltpu.VMEM((RPW,), jnp.int32),
            pltpu.VMEM((RPW,), jnp.int32),
            pltpu.VMEM((RPW,), jnp.int32),
            pltpu.VMEM((HR, 2 * D), jnp.float32),
            pltpu.VMEM((HR, 2 * D), jnp.float32),
            pltpu.VMEM((HR, 2 * D), jnp.float32),
            pltpu.VMEM((L, RPW), jnp.int32),
            pltpu.VMEM((L, RPW), jnp.int32),
            pltpu.VMEM((L, RPW), jnp.int32),
            pltpu.VMEM((HR, 2 * D), jnp.float32),
            pltpu.VMEM((HR, 2 * D), jnp.float32),
            pltpu.VMEM((HR, 2 * D), jnp.float32),
            pltpu.VMEM((NBUF, HR, 2 * D), jnp.float32),
            pltpu.VMEM((NBUF, HR, 2 * D), jnp.float32),
            pltpu.VMEM((NBUF, HR, 2 * D), jnp.float32),
            pltpu.VMEM((NBUF, HR), jnp.int32),
            pltpu.VMEM((NBUF, HR), jnp.int32),
            pltpu.VMEM((NBUF, HR), jnp.int32),
            pltpu.VMEM((1, 128), jnp.float32),
            pltpu.SemaphoreType.DMA,
            pltpu.SemaphoreType.DMA((NBUF,)),
        ],
    )
    partials = run(user_ids, pos_ids, neg_ids, unbr_t, pnbr_t, nnbr_t,
                   utab2, itab2)
    return jnp.sum(partials)


# speedup vs baseline: 1.2032x; 1.1949x over previous
import functools

import jax
import jax.numpy as jnp
from jax import lax
from jax.experimental import pallas as pl
from jax.experimental.pallas import tpu as pltpu
from jax.experimental.pallas import tpu_sc as plsc

NC = 2
NS = 16
NW = NC * NS
B = 4096
D = 64
L = 50
MARGIN = 1.0
RPW = B // NW
PPW = RPW // 2
KG = D // 16
NBUF = 2


def _tcf_body(uid_h, pid_h, nid_h, unbr_h, pnbr_h, nnbr_h, utab_h, itab_h,
              out_h,
              uidx_v, pidx_v, nidx_v, urows_v, prows_v, nrows_v,
              uni_v, pni_v, nni_v, ubag_v, pbag_v, nbag_v, out_v,
              ssem, bsem):
    wid = lax.axis_index("s") * NC + lax.axis_index("c")
    base = wid * RPW
    pbase = wid * PPW

    pltpu.sync_copy(uid_h.at[pl.ds(base, RPW)], uidx_v)
    pltpu.sync_copy(pid_h.at[pl.ds(base, RPW)], pidx_v)
    pltpu.sync_copy(nid_h.at[pl.ds(base, RPW)], nidx_v)
    cu = pltpu.async_copy(utab_h.at[uidx_v], urows_v, ssem)
    cp = pltpu.async_copy(itab_h.at[pidx_v], prows_v, ssem)
    cn = pltpu.async_copy(itab_h.at[nidx_v], nrows_v, ssem)
    pltpu.sync_copy(unbr_h.at[pl.ds(pbase, PPW)], uni_v)
    pltpu.sync_copy(pnbr_h.at[pl.ds(pbase, PPW)], pni_v)
    pltpu.sync_copy(nnbr_h.at[pl.ds(pbase, PPW)], nni_v)

    def start_pair(p):
        slot = lax.rem(p, NBUF)
        pltpu.async_copy(itab_h.at[uni_v.at[p]], ubag_v.at[slot],
                         bsem.at[slot])
        pltpu.async_copy(utab_h.at[pni_v.at[p]], pbag_v.at[slot],
                         bsem.at[slot])
        pltpu.async_copy(utab_h.at[nni_v.at[p]], nbag_v.at[slot],
                         bsem.at[slot])

    def wait_pair(p):
        slot = lax.rem(p, NBUF)
        pltpu.make_async_copy(itab_h.at[uni_v.at[p]], ubag_v.at[slot],
                              bsem.at[slot]).wait()
        pltpu.make_async_copy(utab_h.at[pni_v.at[p]], pbag_v.at[slot],
                              bsem.at[slot]).wait()
        pltpu.make_async_copy(utab_h.at[nni_v.at[p]], nbag_v.at[slot],
                              bsem.at[slot]).wait()

    for p in range(NBUF - 1):
        start_pair(p)
    cu.wait()
    cp.wait()
    cn.wait()

    inv_l = jnp.float32(1.0 / L)
    zero = jnp.zeros((16,), jnp.float32)

    def pair_body(p, acc):
        @pl.when(p + (NBUF - 1) < PPW)
        def _():
            start_pair(p + (NBUF - 1))

        wait_pair(p)
        slot = lax.rem(p, NBUF)
        for r in range(2):
            def red(j, c):
                outs = []
                for t, bag in enumerate((ubag_v, pbag_v, nbag_v)):
                    for k in range(KG):
                        outs.append(c[t * KG + k]
                                    + bag[slot, r * L + j, pl.ds(k * 16, 16)])
                return tuple(outs)

            sums = lax.fori_loop(0, L, red, (zero,) * (3 * KG))
            row = p * 2 + r
            new = []
            for k in range(KG):
                ub = sums[k] * inv_l
                pb = sums[KG + k] * inv_l
                nb = sums[2 * KG + k] * inv_l
                u = urows_v[row, pl.ds(k * 16, 16)]
                pe = prows_v[row, pl.ds(k * 16, 16)]
                ne = nrows_v[row, pl.ds(k * 16, 16)]
                tpos = u + ub * pb - pe
                tneg = u + ub * nb - ne
                v = MARGIN + tpos * tpos - tneg * tneg
                new.append(acc[k] + jnp.maximum(v, 0.0))
            acc = tuple(new)
        return acc

    acc = lax.fori_loop(0, PPW, pair_body, (zero,) * KG)
    out_v[0, :] = acc[0] + acc[1] + acc[2] + acc[3]
    pltpu.sync_copy(out_v, out_h.at[pl.ds(wid, 1)])


def kernel(user_ids, pos_ids, neg_ids, user_nbr_items, pos_item_nbr_users,
           neg_item_nbr_users, user_table, item_table):
    uid = user_ids.astype(jnp.int32)
    pid = pos_ids.astype(jnp.int32)
    nid = neg_ids.astype(jnp.int32)
    unbr = user_nbr_items.astype(jnp.int32).reshape(B // 2, 2 * L)
    pnbr = pos_item_nbr_users.astype(jnp.int32).reshape(B // 2, 2 * L)
    nnbr = neg_item_nbr_users.astype(jnp.int32).reshape(B // 2, 2 * L)

    mesh = plsc.VectorSubcoreMesh(core_axis_name="c", subcore_axis_name="s")
    run = pl.kernel(
        _tcf_body,
        mesh=mesh,
        compiler_params=pltpu.CompilerParams(use_tc_tiling_on_sc=False),
        out_type=jax.ShapeDtypeStruct((NW, 16), jnp.float32),
        scratch_types=[
            pltpu.VMEM((RPW,), jnp.int32),
            pltpu.VMEM((RPW,), jnp.int32),
            pltpu.VMEM((RPW,), jnp.int32),
            pltpu.VMEM((RPW, D), jnp.float32),
            pltpu.VMEM((RPW, D), jnp.float32),
            pltpu.VMEM((RPW, D), jnp.float32),
            pltpu.VMEM((PPW, 2 * L), jnp.int32),
            pltpu.VMEM((PPW, 2 * L), jnp.int32),
            pltpu.VMEM((PPW, 2 * L), jnp.int32),
            pltpu.VMEM((NBUF, 2 * L, D), jnp.float32),
            pltpu.VMEM((NBUF, 2 * L, D), jnp.float32),
            pltpu.VMEM((NBUF, 2 * L, D), jnp.float32),
            pltpu.VMEM((1, 16), jnp.float32),
            pltpu.SemaphoreType.DMA,
            pltpu.SemaphoreType.DMA((NBUF,)),
        ],
    )
    partials = run(uid, pid, nid, unbr, pnbr, nnbr, user_table, item_table)
    return jnp.sum(partials)
